# baseline (speedup 1.0000x reference)
import functools

import jax
import jax.numpy as jnp
from jax import lax
from jax.experimental import pallas as pl
from jax.experimental.pallas import tpu as pltpu
from jax.experimental.pallas import tpu_sc as plsc

_NC = 2
_NS = 16
_NW = _NC * _NS
_LANES = 16


def kernel(in_idx, tok_table, pos_table):
    B, S = in_idx.shape
    V, E = tok_table.shape
    PW = S // _NW
    CH = 16
    CPB = PW // CH
    NCH = B * CPB
    NBUF = 4
    EL = E // _LANES
    assert S % _NW == 0 and PW % CH == 0 and E % _LANES == 0
    assert CPB == NBUF

    if in_idx.dtype != jnp.int32:
        in_idx = in_idx.astype(jnp.int32)

    mesh = plsc.VectorSubcoreMesh(core_axis_name="c", subcore_axis_name="s")

    @functools.partial(
        pl.kernel,
        out_type=jax.ShapeDtypeStruct((B, S, E), jnp.float32),
        mesh=mesh,
        scratch_types=(
            [pltpu.VMEM((B, PW), jnp.int32),
             pltpu.VMEM((PW, E), jnp.float32)]
            + [pltpu.VMEM((CH, E), jnp.float32)] * NBUF
            + [pltpu.SemaphoreType.DMA] * (1 + 2 * NBUF)
        ),
    )
    def emb_kernel(idx_hbm, tok_hbm, pos_hbm, out_hbm, idx_v, pos_v, *rest):
        tok = list(rest[:NBUF])
        psem = rest[NBUF]
        gsem = list(rest[NBUF + 1:2 * NBUF + 1])
        osem = list(rest[2 * NBUF + 1:])
        wid = lax.axis_index("s") * _NC + lax.axis_index("c")
        p0 = wid * PW

        idx_cps = [
            pltpu.async_copy(idx_hbm.at[b, pl.ds(p0, PW)], idx_v.at[b], psem)
            for b in range(B)]
        pos_cp = pltpu.async_copy(pos_hbm.at[pl.ds(p0, PW)], pos_v, psem)

        def gather_copy(g, buf):
            bg = lax.div(g, CPB) if not isinstance(g, int) else g // CPB
            pr = (lax.rem(g, CPB) if not isinstance(g, int) else g % CPB) * CH
            return pltpu.make_async_copy(
                tok_hbm.at[idx_v.at[bg, pl.ds(pr, CH)]], tok[buf], gsem[buf])

        for cp in idx_cps:
            cp.wait()
        gather_copy(0, 0).start()
        gather_copy(1, 1).start()
        pos_cp.wait()

        def group_body(grp, carry):
            for b in range(NBUF):
                g2 = grp * NBUF + b + 2

                @pl.when(g2 < NCH)
                def _issue_ahead():
                    nb = (b + 2) % NBUF

                    @pl.when(g2 >= NBUF)
                    def _drain_writeback():
                        pltpu.make_async_copy(
                            tok[nb], out_hbm.at[0, pl.ds(0, CH)],
                            osem[nb]).wait()

                    gather_copy(g2, nb).start()

                gather_copy(grp * NBUF + b, b).wait()
                tbuf = tok[b]
                prow = b * CH

                @plsc.parallel_loop(0, CH, unroll=2)
                def _add_row(r):
                    for j in range(EL):
                        sl = pl.ds(j * _LANES, _LANES)
                        plsc.addupdate(tbuf.at[r, sl], pos_v[prow + r, sl])

                pltpu.async_copy(
                    tbuf, out_hbm.at[grp, pl.ds(p0 + prow, CH)], osem[b])
            return carry

        lax.fori_loop(0, B, group_body, 0)
        for b in range(NBUF):
            pltpu.make_async_copy(
                tok[b], out_hbm.at[0, pl.ds(0, CH)], osem[b]).wait()

    return emb_kernel(in_idx, tok_table, pos_table)


# speedup vs baseline: 1.1830x; 1.1830x over previous
import functools

import jax
import jax.numpy as jnp
from jax import lax
from jax.experimental import pallas as pl
from jax.experimental.pallas import tpu as pltpu
from jax.experimental.pallas import tpu_sc as plsc

_NC = 2
_NS = 16
_NW = _NC * _NS
_LANES = 16


def kernel(in_idx, tok_table, pos_table):
    B, S = in_idx.shape
    V, E = tok_table.shape
    PW = S // _NW
    CH = 16
    CPB = PW // CH
    NCH = B * CPB
    NBUF = 4
    EL = E // _LANES
    assert S % _NW == 0 and PW % CH == 0 and E % _LANES == 0
    assert CPB == NBUF

    if in_idx.dtype != jnp.int32:
        in_idx = in_idx.astype(jnp.int32)

    mesh = plsc.VectorSubcoreMesh(core_axis_name="c", subcore_axis_name="s")

    @functools.partial(
        pl.kernel,
        out_type=jax.ShapeDtypeStruct((B, S, E), jnp.float32),
        mesh=mesh,
        scratch_types=(
            [pltpu.VMEM((B, PW), jnp.int32),
             pltpu.VMEM((PW, E), jnp.float32)]
            + [pltpu.VMEM((NBUF, CH, E), jnp.float32),
               pltpu.SemaphoreType.DMA,
               pltpu.SemaphoreType.DMA((NBUF,)),
               pltpu.SemaphoreType.DMA((NBUF,))]
        ),
    )
    def emb_kernel(idx_hbm, tok_hbm, pos_hbm, out_hbm, idx_v, pos_v,
                   tok_v, psem, gsem, osem):
        wid = lax.axis_index("s") * _NC + lax.axis_index("c")
        p0 = wid * PW

        idx_cps = [
            pltpu.async_copy(idx_hbm.at[b, pl.ds(p0, PW)], idx_v.at[b], psem)
            for b in range(B)]
        pos_cp = pltpu.async_copy(pos_hbm.at[pl.ds(p0, PW)], pos_v, psem)

        def gather_copy(g):
            if isinstance(g, int):
                bg, buf = g // CPB, g % NBUF
            else:
                bg, buf = lax.div(g, CPB), lax.rem(g, NBUF)
            return pltpu.make_async_copy(
                tok_hbm.at[idx_v.at[bg, pl.ds(buf * CH, CH)]],
                tok_v.at[buf], gsem.at[buf])

        def out_copy(g):
            if isinstance(g, int):
                bg, buf = g // CPB, g % NBUF
            else:
                bg, buf = lax.div(g, CPB), lax.rem(g, NBUF)
            return pltpu.make_async_copy(
                tok_v.at[buf], out_hbm.at[bg, pl.ds(p0 + buf * CH, CH)],
                osem.at[buf])

        for cp in idx_cps:
            cp.wait()
        gather_copy(0).start()
        gather_copy(1).start()
        pos_cp.wait()

        def chunk_body(g, carry):
            g2 = g + 2

            @pl.when(g2 < NCH)
            def _issue_ahead():
                @pl.when(g2 >= NBUF)
                def _drain_writeback():
                    out_copy(g2 - NBUF).wait()

                gather_copy(g2).start()

            gather_copy(g).wait()
            buf = lax.rem(g, NBUF)
            prow = buf * CH

            @plsc.parallel_loop(0, CH)
            def _add_row(r):
                for j in range(EL):
                    sl = pl.ds(j * _LANES, _LANES)
                    plsc.addupdate(tok_v.at[buf, r, sl], pos_v[prow + r, sl])

            out_copy(g).start()
            return carry

        lax.fori_loop(0, NCH, chunk_body, 0)
        for g in range(NCH - NBUF, NCH):
            out_copy(g).wait()

    return emb_kernel(in_idx, tok_table, pos_table)


# speedup vs baseline: 1.1840x; 1.0008x over previous
import functools

import jax
import jax.numpy as jnp
from jax import lax
from jax.experimental import pallas as pl
from jax.experimental.pallas import tpu as pltpu
from jax.experimental.pallas import tpu_sc as plsc

_NC = 2
_NS = 16
_NW = _NC * _NS
_LANES = 16


def kernel(in_idx, tok_table, pos_table):
    B, S = in_idx.shape
    V, E = tok_table.shape
    PW = S // _NW
    CH = 16
    CPB = PW // CH
    NCH = B * CPB
    NBUF = 6
    PF = 4
    EL = E // _LANES
    assert S % _NW == 0 and PW % CH == 0 and E % _LANES == 0

    if in_idx.dtype != jnp.int32:
        in_idx = in_idx.astype(jnp.int32)

    mesh = plsc.VectorSubcoreMesh(core_axis_name="c", subcore_axis_name="s")

    @functools.partial(
        pl.kernel,
        out_type=jax.ShapeDtypeStruct((B, S, E), jnp.float32),
        mesh=mesh,
        scratch_types=(
            [pltpu.VMEM((B, PW), jnp.int32),
             pltpu.VMEM((PW, E), jnp.float32)]
            + [pltpu.VMEM((NBUF, CH, E), jnp.float32),
               pltpu.SemaphoreType.DMA,
               pltpu.SemaphoreType.DMA((NBUF,)),
               pltpu.SemaphoreType.DMA((NBUF,))]
        ),
    )
    def emb_kernel(idx_hbm, tok_hbm, pos_hbm, out_hbm, idx_v, pos_v,
                   tok_v, psem, gsem, osem):
        wid = lax.axis_index("s") * _NC + lax.axis_index("c")
        p0 = wid * PW

        idx_cps = [
            pltpu.async_copy(idx_hbm.at[b, pl.ds(p0, PW)], idx_v.at[b], psem)
            for b in range(B)]
        pos_cp = pltpu.async_copy(pos_hbm.at[pl.ds(p0, PW)], pos_v, psem)

        def gather_copy(g):
            if isinstance(g, int):
                bg, pc, buf = g // CPB, g % CPB, g % NBUF
            else:
                bg, pc, buf = lax.div(g, CPB), lax.rem(g, CPB), lax.rem(g, NBUF)
            return pltpu.make_async_copy(
                tok_hbm.at[idx_v.at[bg, pl.ds(pc * CH, CH)]],
                tok_v.at[buf], gsem.at[buf])

        def out_copy(g):
            if isinstance(g, int):
                bg, pc, buf = g // CPB, g % CPB, g % NBUF
            else:
                bg, pc, buf = lax.div(g, CPB), lax.rem(g, CPB), lax.rem(g, NBUF)
            return pltpu.make_async_copy(
                tok_v.at[buf], out_hbm.at[bg, pl.ds(p0 + pc * CH, CH)],
                osem.at[buf])

        for cp in idx_cps:
            cp.wait()
        for g in range(PF):
            gather_copy(g).start()
        pos_cp.wait()

        def chunk_body(g, carry):
            g2 = g + PF

            @pl.when(g2 < NCH)
            def _issue_ahead():
                @pl.when(g2 >= NBUF)
                def _drain_writeback():
                    out_copy(g2 - NBUF).wait()

                gather_copy(g2).start()

            gather_copy(g).wait()
            prow = lax.rem(g, CPB) * CH
            buf = lax.rem(g, NBUF)

            @plsc.parallel_loop(0, CH)
            def _add_row(r):
                for j in range(EL):
                    sl = pl.ds(j * _LANES, _LANES)
                    plsc.addupdate(tok_v.at[buf, r, sl], pos_v[prow + r, sl])

            out_copy(g).start()
            return carry

        lax.fori_loop(0, NCH, chunk_body, 0)
        for g in range(NCH - NBUF, NCH):
            out_copy(g).wait()

    return emb_kernel(in_idx, tok_table, pos_table)
